# speedup vs baseline: 2.2657x; 2.2657x over previous
import functools

import jax
import jax.numpy as jnp
from jax import lax
from jax.experimental import pallas as pl
from jax.experimental.pallas import tpu as pltpu
from jax.experimental.pallas import tpu_sc as plsc

EPS = 1e-12

B, S, H = 4, 2048, 768
N = B * S
NC, NS = 2, 16
NW = NC * NS
ROWS_PER_TILE = N // NW
CHUNK = 64
NCHUNK = ROWS_PER_TILE // CHUNK

TC_ROWS = 1024
TC_GRID = N // TC_ROWS
POS_BLOCKS_PER_SEQ = S // TC_ROWS


def _sc_gather(word_table, flat_ids):
    mesh = plsc.VectorSubcoreMesh(core_axis_name="c", subcore_axis_name="s")
    ids2d = flat_ids.reshape(NW * NCHUNK, CHUNK)

    @functools.partial(
        pl.kernel,
        mesh=mesh,
        out_type=jax.ShapeDtypeStruct((N, H), word_table.dtype),
        scratch_types=[
            pltpu.VMEM((NCHUNK, CHUNK), jnp.int32),
            pltpu.VMEM((CHUNK, H), jnp.float32),
            pltpu.SemaphoreType.DMA,
        ],
    )
    def k(table_hbm, idx_hbm, out_hbm, idx_v, rows_v, sem):
        wid = lax.axis_index("s") * NC + lax.axis_index("c")
        pltpu.sync_copy(idx_hbm.at[pl.ds(wid * NCHUNK, NCHUNK)], idx_v)
        for c in range(NCHUNK):
            pltpu.async_copy(table_hbm.at[idx_v.at[c]], rows_v, sem).wait()
            base = (wid * NCHUNK + c) * CHUNK
            pltpu.sync_copy(rows_v, out_hbm.at[pl.ds(base, CHUNK)])

    return k(word_table, ids2d)


def _ln_body(g_ref, p_ref, tt_ref, ty_ref, w_ref, b_ref, o_ref):
    tt = tt_ref[...]
    t0 = ty_ref[0:1, :]
    t1 = ty_ref[1:2, :]
    e = g_ref[...] + p_ref[...] + t0 + tt * (t1 - t0)
    u = jnp.mean(e, axis=-1, keepdims=True)
    d = e - u
    s = jnp.mean(d * d, axis=-1, keepdims=True)
    x = d * lax.rsqrt(s + EPS)
    o_ref[...] = w_ref[...] * x + b_ref[...]


def _tc_add_ln(gathered, pos_table, tt_col, type_table, ln_weight, ln_bias):
    return pl.pallas_call(
        _ln_body,
        grid=(TC_GRID,),
        in_specs=[
            pl.BlockSpec((TC_ROWS, H), lambda i: (i, 0)),
            pl.BlockSpec((TC_ROWS, H), lambda i: (i % POS_BLOCKS_PER_SEQ, 0)),
            pl.BlockSpec((TC_ROWS, 1), lambda i: (i, 0)),
            pl.BlockSpec((2, H), lambda i: (0, 0)),
            pl.BlockSpec((1, H), lambda i: (0, 0)),
            pl.BlockSpec((1, H), lambda i: (0, 0)),
        ],
        out_specs=pl.BlockSpec((TC_ROWS, H), lambda i: (i, 0)),
        out_shape=jax.ShapeDtypeStruct((N, H), jnp.float32),
    )(gathered, pos_table, tt_col, type_table, ln_weight, ln_bias)


def kernel(input_ids, token_type_ids, word_table, pos_table, type_table,
           ln_weight, ln_bias):
    flat_ids = input_ids.reshape(N).astype(jnp.int32)
    gathered = _sc_gather(word_table, flat_ids)
    tt_col = token_type_ids.reshape(N, 1).astype(jnp.float32)
    out = _tc_add_ln(gathered, pos_table, tt_col, type_table,
                     ln_weight.reshape(1, H), ln_bias.reshape(1, H))
    return out.reshape(B, S, H)


# speedup vs baseline: 2.3467x; 1.0358x over previous
import functools

import jax
import jax.numpy as jnp
from jax import lax
from jax.experimental import pallas as pl
from jax.experimental.pallas import tpu as pltpu
from jax.experimental.pallas import tpu_sc as plsc

EPS = 1e-12

B, S, H = 4, 2048, 768
N = B * S
NC, NS = 2, 16
NW = NC * NS
ROWS_PER_TILE = N // NW
CHUNK = 64
NCHUNK = ROWS_PER_TILE // CHUNK

TC_ROWS = 1024
TC_GRID = N // TC_ROWS
POS_BLOCKS_PER_SEQ = S // TC_ROWS


def _sc_gather(word_table, flat_ids):
    mesh = plsc.VectorSubcoreMesh(core_axis_name="c", subcore_axis_name="s")
    ids2d = flat_ids.reshape(NW * NCHUNK, CHUNK)

    @functools.partial(
        pl.kernel,
        mesh=mesh,
        out_type=jax.ShapeDtypeStruct((N, H), word_table.dtype),
        scratch_types=[
            pltpu.VMEM((NCHUNK, CHUNK), jnp.int32),
            pltpu.VMEM((CHUNK, H), jnp.float32),
            pltpu.VMEM((CHUNK, H), jnp.float32),
            pltpu.SemaphoreType.DMA,
        ],
    )
    def k(table_hbm, idx_hbm, out_hbm, idx_v, rows_a, rows_b, sem):
        wid = lax.axis_index("s") * NC + lax.axis_index("c")
        pltpu.sync_copy(idx_hbm.at[pl.ds(wid * NCHUNK, NCHUNK)], idx_v)
        bufs = (rows_a, rows_b)
        descs = [None] * NCHUNK
        for c in range(min(2, NCHUNK)):
            descs[c] = pltpu.async_copy(table_hbm.at[idx_v.at[c]], bufs[c % 2], sem)
        for c in range(NCHUNK):
            descs[c].wait()
            base = (wid * NCHUNK + c) * CHUNK
            pltpu.sync_copy(bufs[c % 2], out_hbm.at[pl.ds(base, CHUNK)])
            if c + 2 < NCHUNK:
                descs[c + 2] = pltpu.async_copy(
                    table_hbm.at[idx_v.at[c + 2]], bufs[c % 2], sem)

    return k(word_table, ids2d)


def _ln_body(g_ref, p_ref, tt_ref, ty_ref, w_ref, b_ref, o_ref):
    tt = tt_ref[...]
    t0 = ty_ref[0:1, :]
    t1 = ty_ref[1:2, :]
    e = g_ref[...] + p_ref[...] + t0 + tt * (t1 - t0)
    u = jnp.mean(e, axis=-1, keepdims=True)
    d = e - u
    s = jnp.mean(d * d, axis=-1, keepdims=True)
    x = d * lax.rsqrt(s + EPS)
    o_ref[...] = w_ref[...] * x + b_ref[...]


def _tc_add_ln(gathered, pos_table, tt_col, type_table, ln_weight, ln_bias):
    return pl.pallas_call(
        _ln_body,
        grid=(TC_GRID,),
        in_specs=[
            pl.BlockSpec((TC_ROWS, H), lambda i: (i, 0)),
            pl.BlockSpec((TC_ROWS, H), lambda i: (i % POS_BLOCKS_PER_SEQ, 0)),
            pl.BlockSpec((TC_ROWS, 1), lambda i: (i, 0)),
            pl.BlockSpec((2, H), lambda i: (0, 0)),
            pl.BlockSpec((1, H), lambda i: (0, 0)),
            pl.BlockSpec((1, H), lambda i: (0, 0)),
        ],
        out_specs=pl.BlockSpec((TC_ROWS, H), lambda i: (i, 0)),
        out_shape=jax.ShapeDtypeStruct((N, H), jnp.float32),
    )(gathered, pos_table, tt_col, type_table, ln_weight, ln_bias)


def kernel(input_ids, token_type_ids, word_table, pos_table, type_table,
           ln_weight, ln_bias):
    flat_ids = input_ids.reshape(N).astype(jnp.int32)
    gathered = _sc_gather(word_table, flat_ids)
    tt_col = token_type_ids.reshape(N, 1).astype(jnp.float32)
    out = _tc_add_ln(gathered, pos_table, tt_col, type_table,
                     ln_weight.reshape(1, H), ln_bias.reshape(1, H))
    return out.reshape(B, S, H)


# speedup vs baseline: 2.5031x; 1.0667x over previous
import functools

import jax
import jax.numpy as jnp
from jax import lax
from jax.experimental import pallas as pl
from jax.experimental.pallas import tpu as pltpu
from jax.experimental.pallas import tpu_sc as plsc

EPS = 1e-12

B, S, H = 4, 2048, 768
N = B * S
NC, NS = 2, 16
NW = NC * NS
ROWS_PER_TILE = N // NW
CHUNK = 64
NCHUNK = ROWS_PER_TILE // CHUNK

TC_ROWS = S
TC_GRID = N // TC_ROWS


def _sc_gather(word_table, flat_ids):
    mesh = plsc.VectorSubcoreMesh(core_axis_name="c", subcore_axis_name="s")
    ids2d = flat_ids.reshape(NW * NCHUNK, CHUNK)

    @functools.partial(
        pl.kernel,
        mesh=mesh,
        out_type=jax.ShapeDtypeStruct((N, H), word_table.dtype),
        scratch_types=[
            pltpu.VMEM((NCHUNK, CHUNK), jnp.int32),
            pltpu.VMEM((CHUNK, H), jnp.float32),
            pltpu.VMEM((CHUNK, H), jnp.float32),
            pltpu.SemaphoreType.DMA,
        ],
    )
    def k(table_hbm, idx_hbm, out_hbm, idx_v, rows_a, rows_b, sem):
        wid = lax.axis_index("s") * NC + lax.axis_index("c")
        pltpu.sync_copy(idx_hbm.at[pl.ds(wid * NCHUNK, NCHUNK)], idx_v)
        bufs = (rows_a, rows_b)
        descs = [None] * NCHUNK
        for c in range(min(2, NCHUNK)):
            descs[c] = pltpu.async_copy(table_hbm.at[idx_v.at[c]], bufs[c % 2], sem)
        for c in range(NCHUNK):
            descs[c].wait()
            base = (wid * NCHUNK + c) * CHUNK
            pltpu.sync_copy(bufs[c % 2], out_hbm.at[pl.ds(base, CHUNK)])
            if c + 2 < NCHUNK:
                descs[c + 2] = pltpu.async_copy(
                    table_hbm.at[idx_v.at[c + 2]], bufs[c % 2], sem)

    return k(word_table, ids2d)


def _ln_body(g_ref, p_ref, tt_ref, ty_ref, w_ref, b_ref, o_ref):
    tt = tt_ref[...]
    t0 = ty_ref[0:1, :]
    t1 = ty_ref[1:2, :]
    e = g_ref[...] + p_ref[...] + t0 + tt * (t1 - t0)
    u = jnp.mean(e, axis=-1, keepdims=True)
    d = e - u
    s = jnp.mean(d * d, axis=-1, keepdims=True)
    x = d * lax.rsqrt(s + EPS)
    o_ref[...] = w_ref[...] * x + b_ref[...]


def _tc_add_ln(gathered, pos_table, tt_col, type_table, ln_weight, ln_bias):
    return pl.pallas_call(
        _ln_body,
        grid=(TC_GRID,),
        in_specs=[
            pl.BlockSpec((TC_ROWS, H), lambda i: (i, 0)),
            pl.BlockSpec((TC_ROWS, H), lambda i: (0, 0)),
            pl.BlockSpec((TC_ROWS, 1), lambda i: (i, 0)),
            pl.BlockSpec((2, H), lambda i: (0, 0)),
            pl.BlockSpec((1, H), lambda i: (0, 0)),
            pl.BlockSpec((1, H), lambda i: (0, 0)),
        ],
        out_specs=pl.BlockSpec((TC_ROWS, H), lambda i: (i, 0)),
        out_shape=jax.ShapeDtypeStruct((N, H), jnp.float32),
    )(gathered, pos_table, tt_col, type_table, ln_weight, ln_bias)


def kernel(input_ids, token_type_ids, word_table, pos_table, type_table,
           ln_weight, ln_bias):
    flat_ids = input_ids.reshape(N).astype(jnp.int32)
    gathered = _sc_gather(word_table, flat_ids)
    tt_col = token_type_ids.reshape(N, 1).astype(jnp.float32)
    out = _tc_add_ln(gathered, pos_table, tt_col, type_table,
                     ln_weight.reshape(1, H), ln_bias.reshape(1, H))
    return out.reshape(B, S, H)


# speedup vs baseline: 2.5055x; 1.0010x over previous
import functools

import jax
import jax.numpy as jnp
from jax import lax
from jax.experimental import pallas as pl
from jax.experimental.pallas import tpu as pltpu
from jax.experimental.pallas import tpu_sc as plsc

EPS = 1e-12

B, S, H = 4, 2048, 768
N = B * S
NC, NS = 2, 16
NW = NC * NS
ROWS_PER_TILE = N // NW
CHUNK = 64
NCHUNK = ROWS_PER_TILE // CHUNK

TC_ROWS = S
TC_GRID = N // TC_ROWS


def _sc_gather(word_table, flat_ids):
    mesh = plsc.VectorSubcoreMesh(core_axis_name="c", subcore_axis_name="s")
    ids2d = flat_ids.reshape(NW * NCHUNK, CHUNK)

    @functools.partial(
        pl.kernel,
        mesh=mesh,
        out_type=jax.ShapeDtypeStruct((N, H), word_table.dtype),
        scratch_types=[
            pltpu.VMEM((NCHUNK, CHUNK), jnp.int32),
            pltpu.VMEM((CHUNK, H), jnp.float32),
            pltpu.VMEM((CHUNK, H), jnp.float32),
            pltpu.SemaphoreType.DMA,
        ],
    )
    def k(table_hbm, idx_hbm, out_hbm, idx_v, rows_a, rows_b, sem):
        wid = lax.axis_index("s") * NC + lax.axis_index("c")
        pltpu.sync_copy(idx_hbm.at[pl.ds(wid * NCHUNK, NCHUNK)], idx_v)
        bufs = (rows_a, rows_b)
        descs = [None] * NCHUNK
        for c in range(min(2, NCHUNK)):
            descs[c] = pltpu.async_copy(table_hbm.at[idx_v.at[c]], bufs[c % 2], sem)
        for c in range(NCHUNK):
            descs[c].wait()
            base = (wid * NCHUNK + c) * CHUNK
            pltpu.sync_copy(bufs[c % 2], out_hbm.at[pl.ds(base, CHUNK)])
            if c + 2 < NCHUNK:
                descs[c + 2] = pltpu.async_copy(
                    table_hbm.at[idx_v.at[c + 2]], bufs[c % 2], sem)

    return k(word_table, ids2d)


def _ln_body(g_ref, p_ref, tt_ref, ty_ref, w_ref, b_ref, o_ref):
    tt = tt_ref[...]
    t0 = ty_ref[0:1, :]
    t1 = ty_ref[1:2, :]
    e = g_ref[...] + p_ref[...] + t0 + tt * (t1 - t0)
    u = jnp.mean(e, axis=-1, keepdims=True)
    d = e - u
    s = jnp.mean(d * d, axis=-1, keepdims=True)
    x = d * lax.rsqrt(s + EPS)
    o_ref[...] = w_ref[...] * x + b_ref[...]


def _tc_add_ln(gathered, pos_table, tt_col, type_table, ln_weight, ln_bias):
    return pl.pallas_call(
        _ln_body,
        grid=(TC_GRID,),
        in_specs=[
            pl.BlockSpec((TC_ROWS, H), lambda i: (i, 0)),
            pl.BlockSpec((TC_ROWS, H), lambda i: (0, 0)),
            pl.BlockSpec((TC_ROWS, 1), lambda i: (i, 0)),
            pl.BlockSpec((2, H), lambda i: (0, 0)),
            pl.BlockSpec((1, H), lambda i: (0, 0)),
            pl.BlockSpec((1, H), lambda i: (0, 0)),
        ],
        out_specs=pl.BlockSpec((TC_ROWS, H), lambda i: (i, 0)),
        out_shape=jax.ShapeDtypeStruct((N, H), jnp.float32),
        compiler_params=pltpu.CompilerParams(
            dimension_semantics=("parallel",)),
    )(gathered, pos_table, tt_col, type_table, ln_weight, ln_bias)


def kernel(input_ids, token_type_ids, word_table, pos_table, type_table,
           ln_weight, ln_bias):
    flat_ids = input_ids.reshape(N).astype(jnp.int32)
    gathered = _sc_gather(word_table, flat_ids)
    tt_col = token_type_ids.reshape(N, 1).astype(jnp.float32)
    out = _tc_add_ln(gathered, pos_table, tt_col, type_table,
                     ln_weight.reshape(1, H), ln_bias.reshape(1, H))
    return out.reshape(B, S, H)
